# baseline (speedup 1.0000x reference)
import functools

import jax
import jax.numpy as jnp
from jax import lax
from jax.experimental import pallas as pl
from jax.experimental.pallas import tpu as pltpu
from jax.experimental.pallas import tpu_sc as plsc

N = 1_600_000
T = 16
G = 10_000
NC, NS = 2, 16
NW = NC * NS
NTILE = N // 128
BT = 16
BLOCK = BT * 128
NGRP = BLOCK // 16
NFULL = NTILE // BT
TOTAL_BLKS = NFULL + 1
SLOTS = -(-TOTAL_BLKS // NW)
TAIL_TS = NTILE - BT
TAIL_G0 = 96
GP = 10_240
DUMP = G
XW = BT * 1024


def _bcast(v, j):
    return v.at[jnp.full((T,), j, jnp.int32)].get(mode="promise_in_bounds")


def _treesum(vals):
    vals = list(vals)
    while len(vals) > 1:
        nxt = [a + b for a, b in zip(vals[0::2], vals[1::2])]
        if len(vals) % 2:
            nxt.append(vals[-1])
        vals = nxt
    return vals[0]


def _sc_partials(x2, k2, e2, i2, x3, k3, e3, i3):
    mesh = plsc.VectorSubcoreMesh(core_axis_name="c", subcore_axis_name="s")

    @functools.partial(
        pl.kernel,
        mesh=mesh,
        out_type=jax.ShapeDtypeStruct((NC, G, T), jnp.float32),
        compiler_params=pltpu.CompilerParams(
            use_tc_tiling_on_sc=False, needs_layout_passes=False),
        scratch_types=[
            pltpu.VMEM((2 * XW,), jnp.float32),
            pltpu.VMEM((2 * XW,), jnp.float32),
            pltpu.VMEM((BT, 128), jnp.float32),
            pltpu.VMEM((BT, 128), jnp.float32),
            pltpu.VMEM((BT, 128), jnp.float32),
            pltpu.VMEM((BT, 128), jnp.float32),
            pltpu.VMEM((BLOCK,), jnp.int32),
            pltpu.VMEM((BLOCK,), jnp.int32),
            pltpu.VMEM((16, T), jnp.float32),
            pltpu.VMEM((T,), jnp.float32),
            pltpu.VMEM((T, 16), jnp.float32),
            pltpu.VMEM((16,), jnp.int32),
            pltpu.SMEM((2,), jnp.int32),
            pltpu.SemaphoreType.DMA((2,)),
            pltpu.VMEM_SHARED((GP, T), jnp.float32),
        ],
    )
    def k(x2_h, k2_h, e2_h, i2_h, x3_h, k3_h, e3_h, i3_h, out_h,
          xb0, xb1, kb0, kb1, eb0, eb1, jb0, jb1, vb, accr, accT, gvr, scr,
          sems, acc_sh):
        bufs = ((xb0, kb0, eb0, jb0), (xb1, kb1, eb1, jb1))
        c = lax.axis_index("c")
        s = lax.axis_index("s")
        wid = c * NS + s

        zrows = GP // NS
        for r in range(16):
            vb[r, :] = jnp.zeros((T,), jnp.float32)
        for r in range(zrows // 16):
            pltpu.sync_copy(vb, acc_sh.at[pl.ds(s * zrows + r * 16, 16)])
        plsc.subcore_barrier()

        lane = lax.iota(jnp.int32, 16)
        zero16 = jnp.zeros((16,), jnp.int32)
        dump_vec = jnp.full((16,), DUMP, jnp.int32)
        xconst = (lane >> 3) * XW + (lane & 7) * 128
        gvr[...] = dump_vec

        b0 = (wid * TOTAL_BLKS) // NW
        b1 = ((wid + 1) * TOTAL_BLKS) // NW

        def _slot_blk(j):
            return jnp.where(b0 + j < b1, b0 + j, b0)

        def _tilestart(blk):
            return jnp.where(blk == NFULL, TAIL_TS, blk * BT)

        def _copies(x_h, k_h, e_h, i_h, b, blk):
            ts = _tilestart(blk)
            xb, kb, eb, ib = bufs[b]
            return (
                pltpu.make_async_copy(
                    x_h.at[0, pl.ds(ts * 1024, XW)],
                    xb.at[pl.ds(0, XW)], sems.at[b]),
                pltpu.make_async_copy(
                    x_h.at[1, pl.ds(ts * 1024, XW)],
                    xb.at[pl.ds(XW, XW)], sems.at[b]),
                pltpu.make_async_copy(
                    k_h.at[pl.ds(ts, BT), :], kb, sems.at[b]),
                pltpu.make_async_copy(
                    e_h.at[pl.ds(ts, BT), :], eb, sems.at[b]),
                pltpu.make_async_copy(
                    i_h.at[pl.ds(ts * 128, BLOCK)], ib, sems.at[b]),
            )

        def start_blk(x_h, k_h, e_h, i_h, b, j):
            for cp in _copies(x_h, k_h, e_h, i_h, b, _slot_blk(j)):
                cp.start()

        def wait_blk(x_h, k_h, e_h, i_h, b):
            for cp in _copies(x_h, k_h, e_h, i_h, b, 0):
                cp.wait()

        def scan_block(b, j):
            blk = b0 + j
            valid = blk < b1
            gstart = jnp.where(
                valid, jnp.where(blk == NFULL, TAIL_G0, 0), NGRP)
            xb, kb, eb, ib = bufs[b]
            idg = ib[pl.ds(jnp.minimum(gstart, NGRP - 1) * 16, 16)]
            scr[0] = idg[0]
            scr[1] = 0
            zt = jnp.zeros((T,), jnp.float32)
            accr[...] = zt
            for t in range(16):
                accT[t, :] = zt

            def collapse():
                cols = []
                for sl in range(16):
                    cols.append(plsc.load_gather(
                        accT, [lane, jnp.full((T,), sl, jnp.int32)]))
                for sl in range(16):
                    accT[sl, :] = zt
                accr[...] = accr[...] + _treesum(cols)

            def group(g2, _):
                base = g2 * 32
                krow = g2 >> 2
                koff = (g2 & 3) * 32
                k16a = kb[krow, pl.ds(koff, 16)] * 0.5
                k16b = kb[krow, pl.ds(koff + 16, 16)] * 0.5
                e16a = eb[krow, pl.ds(koff, 16)]
                e16b = eb[krow, pl.ds(koff + 16, 16)]
                id16a = ib[pl.ds(base, 16)]
                id16b = ib[pl.ds(base + 16, 16)]
                rowoff = krow * 1024 + koff
                last = id16b[15]
                has_boundary = last != scr[0]

                def _xrow(j):
                    return plsc.load_gather(
                        xb, [xconst + (rowoff + j)])

                def fast():
                    for t in range(16):
                        off = rowoff + ((t >> 3) * XW + (t & 7) * 128)
                        x1 = xb[pl.ds(off, 16)]
                        x2 = xb[pl.ds(off + 16, 16)]
                        d1 = x1 - e16a
                        d2 = x2 - e16b
                        accT[t, :] = (accT[t, :] + k16a * (d1 * d1)
                                      + k16b * (d2 * d2))

                def slow():
                    collapse()
                    prev_id = scr[0]
                    cnt = scr[1]
                    acc = accr[...]
                    gvec = gvr[...]
                    for j in range(32):
                        x = _xrow(j)
                        kh = k16a if j < 16 else k16b
                        eh = e16a if j < 16 else e16b
                        idh = id16a if j < 16 else id16b
                        d = x - _bcast(eh, j & 15)
                        u = _bcast(kh, j & 15) * (d * d)
                        gid = idh[j & 15]
                        changed = gid != prev_id
                        slot = cnt & 15

                        @pl.when(changed)
                        def _(acc=acc, slot=slot):
                            vb[slot, :] = acc

                        sel = jnp.where(changed, slot, jnp.int32(-1))
                        gvec = jnp.where(lane == sel, prev_id, gvec)
                        flush = changed & (slot == 15)

                        @pl.when(flush)
                        def _(gvec=gvec):
                            pltpu.sync_copy(vb, acc_sh.at[gvec], add=True)

                        gvec = jnp.where(flush, dump_vec, gvec)
                        acc = jnp.where(changed, u, acc + u)
                        cnt = jnp.where(changed, cnt + 1, cnt)
                        prev_id = gid
                    scr[0] = prev_id
                    scr[1] = cnt
                    accr[...] = acc
                    gvr[...] = gvec

                lax.cond(has_boundary, slow, fast)
                return 0

            lax.fori_loop(gstart // 2, NGRP // 2, group, 0)
            collapse()

            slot = scr[1] & 15
            vb[slot, :] = accr[...]
            gvec = jnp.where(lane == slot, scr[0], gvr[...])
            pltpu.sync_copy(vb, acc_sh.at[gvec], add=True)
            gvr[...] = dump_vec

        def run_ntype(x_h, k_h, e_h, i_h):
            start_blk(x_h, k_h, e_h, i_h, 0, 0)

            def pair(i, _):
                j0 = 2 * i
                start_blk(x_h, k_h, e_h, i_h, 1, j0 + 1)
                wait_blk(x_h, k_h, e_h, i_h, 0)
                scan_block(0, j0)
                start_blk(x_h, k_h, e_h, i_h, 0, j0 + 2)
                wait_blk(x_h, k_h, e_h, i_h, 1)
                scan_block(1, j0 + 1)
                return 0

            lax.fori_loop(0, SLOTS // 2, pair, 0)
            wait_blk(x_h, k_h, e_h, i_h, 0)
            scan_block(0, SLOTS - 1)

        run_ntype(x2_h, k2_h, e2_h, i2_h)
        run_ntype(x3_h, k3_h, e3_h, i3_h)

        plsc.subcore_barrier()
        orows = 624
        pltpu.sync_copy(acc_sh.at[pl.ds(s * orows, orows)],
                        out_h.at[c].at[pl.ds(s * orows, orows)])

        @pl.when(s == NS - 1)
        def _():
            pltpu.sync_copy(acc_sh.at[pl.ds(NS * orows, G - NS * orows)],
                            out_h.at[c].at[pl.ds(NS * orows, G - NS * orows)])

    return k(x2, k2, e2, i2, x3, k3, e3, i3)


def _tc_combine(p0, p1):
    def add_k(a_ref, b_ref, o_ref):
        o_ref[...] = a_ref[...] + b_ref[...]

    return pl.pallas_call(
        add_k,
        out_shape=jax.ShapeDtypeStruct((1250, 128), jnp.float32),
    )(p0, p1)


def _xview(x):
    return jnp.transpose(
        x.reshape(NTILE, 128, 2, 8), (2, 0, 3, 1)).reshape(2, NTILE * 1024)


def kernel(x_n2, k_n2, eq_n2, x_n3, k_n3, eq_n3, n2_graph_ids, n3_graph_ids,
           num_graphs):
    del num_graphs
    x2v = _xview(x_n2)
    x3v = _xview(x_n3)
    k2 = k_n2.reshape(NTILE, 128)
    e2 = eq_n2.reshape(NTILE, 128)
    k3 = k_n3.reshape(NTILE, 128)
    e3 = eq_n3.reshape(NTILE, 128)
    i2 = n2_graph_ids.astype(jnp.int32)
    i3 = n3_graph_ids.astype(jnp.int32)
    partials = _sc_partials(x2v, k2, e2, i2, x3v, k3, e3, i3)
    p0 = partials[0].reshape(1250, 128)
    p1 = partials[1].reshape(1250, 128)
    return _tc_combine(p0, p1).reshape(G, T)


# speedup vs baseline: 1.0114x; 1.0114x over previous
import functools

import jax
import jax.numpy as jnp
from jax import lax
from jax.experimental import pallas as pl
from jax.experimental.pallas import tpu as pltpu
from jax.experimental.pallas import tpu_sc as plsc

N = 1_600_000
T = 16
G = 10_000
NC, NS = 2, 16
NW = NC * NS
NTILE = N // 128
BT = 16
BLOCK = BT * 128
NGRP = BLOCK // 16
NFULL = NTILE // BT
TOTAL_BLKS = NFULL + 1
SLOTS = -(-TOTAL_BLKS // NW)
TAIL_TS = NTILE - BT
TAIL_G0 = 96
GP = 10_240
DUMP = G
XW = BT * 1024


def _bcast(v, j):
    return v.at[jnp.full((T,), j, jnp.int32)].get(mode="promise_in_bounds")


def _treesum(vals):
    vals = list(vals)
    while len(vals) > 1:
        nxt = [a + b for a, b in zip(vals[0::2], vals[1::2])]
        if len(vals) % 2:
            nxt.append(vals[-1])
        vals = nxt
    return vals[0]


def _sc_partials(x2, k2, e2, i2, x3, k3, e3, i3):
    mesh = plsc.VectorSubcoreMesh(core_axis_name="c", subcore_axis_name="s")

    @functools.partial(
        pl.kernel,
        mesh=mesh,
        out_type=jax.ShapeDtypeStruct((NC, G, T), jnp.float32),
        compiler_params=pltpu.CompilerParams(
            use_tc_tiling_on_sc=False, needs_layout_passes=False),
        scratch_types=[
            pltpu.VMEM((2 * XW,), jnp.float32),
            pltpu.VMEM((2 * XW,), jnp.float32),
            pltpu.VMEM((BT, 128), jnp.float32),
            pltpu.VMEM((BT, 128), jnp.float32),
            pltpu.VMEM((BT, 128), jnp.float32),
            pltpu.VMEM((BT, 128), jnp.float32),
            pltpu.VMEM((BLOCK,), jnp.int32),
            pltpu.VMEM((BLOCK,), jnp.int32),
            pltpu.VMEM((16, T), jnp.float32),
            pltpu.VMEM((T,), jnp.float32),
            pltpu.VMEM((T, 16), jnp.float32),
            pltpu.VMEM((16,), jnp.int32),
            pltpu.SMEM((2,), jnp.int32),
            pltpu.SemaphoreType.DMA((2,)),
            pltpu.VMEM_SHARED((GP, T), jnp.float32),
        ],
    )
    def k(x2_h, k2_h, e2_h, i2_h, x3_h, k3_h, e3_h, i3_h, out_h,
          xb0, xb1, kb0, kb1, eb0, eb1, jb0, jb1, vb, accr, accT, gvr, scr,
          sems, acc_sh):
        bufs = ((xb0, kb0, eb0, jb0), (xb1, kb1, eb1, jb1))
        c = lax.axis_index("c")
        s = lax.axis_index("s")
        wid = c * NS + s

        zrows = GP // NS
        for r in range(16):
            vb[r, :] = jnp.zeros((T,), jnp.float32)
        for r in range(zrows // 16):
            pltpu.sync_copy(vb, acc_sh.at[pl.ds(s * zrows + r * 16, 16)])
        plsc.subcore_barrier()

        lane = lax.iota(jnp.int32, 16)
        zero16 = jnp.zeros((16,), jnp.int32)
        dump_vec = jnp.full((16,), DUMP, jnp.int32)
        xconst = (lane >> 3) * XW + (lane & 7) * 128
        gvr[...] = dump_vec

        b0 = (wid * TOTAL_BLKS) // NW
        b1 = ((wid + 1) * TOTAL_BLKS) // NW

        def _slot_blk(j):
            return jnp.where(b0 + j < b1, b0 + j, b0)

        def _tilestart(blk):
            return jnp.where(blk == NFULL, TAIL_TS, blk * BT)

        def _copies(x_h, k_h, e_h, i_h, b, blk):
            ts = _tilestart(blk)
            xb, kb, eb, ib = bufs[b]
            return (
                pltpu.make_async_copy(
                    x_h.at[0, pl.ds(ts * 1024, XW)],
                    xb.at[pl.ds(0, XW)], sems.at[b]),
                pltpu.make_async_copy(
                    x_h.at[1, pl.ds(ts * 1024, XW)],
                    xb.at[pl.ds(XW, XW)], sems.at[b]),
                pltpu.make_async_copy(
                    k_h.at[pl.ds(ts, BT), :], kb, sems.at[b]),
                pltpu.make_async_copy(
                    e_h.at[pl.ds(ts, BT), :], eb, sems.at[b]),
                pltpu.make_async_copy(
                    i_h.at[pl.ds(ts * 128, BLOCK)], ib, sems.at[b]),
            )

        def start_blk(x_h, k_h, e_h, i_h, b, j):
            for cp in _copies(x_h, k_h, e_h, i_h, b, _slot_blk(j)):
                cp.start()

        def wait_blk(x_h, k_h, e_h, i_h, b):
            for cp in _copies(x_h, k_h, e_h, i_h, b, 0):
                cp.wait()

        def scan_block(b, j):
            blk = b0 + j
            valid = blk < b1
            gstart = jnp.where(
                valid, jnp.where(blk == NFULL, TAIL_G0, 0), NGRP)
            xb, kb, eb, ib = bufs[b]
            idg = ib[pl.ds(jnp.minimum(gstart, NGRP - 1) * 16, 16)]
            scr[0] = idg[0]
            scr[1] = 0
            zt = jnp.zeros((T,), jnp.float32)
            accr[...] = zt
            for t in range(16):
                accT[t, :] = zt

            def collapse():
                col = jnp.zeros((T,), jnp.float32)
                for t in range(16):
                    st = jnp.sum(accT[t])
                    col = jnp.where(lane == t, st, col)
                    accT[t, :] = zt
                accr[...] = accr[...] + col

            def group(g2, _):
                base = g2 * 32
                krow = g2 >> 2
                koff = (g2 & 3) * 32
                k16a = kb[krow, pl.ds(koff, 16)] * 0.5
                k16b = kb[krow, pl.ds(koff + 16, 16)] * 0.5
                e16a = eb[krow, pl.ds(koff, 16)]
                e16b = eb[krow, pl.ds(koff + 16, 16)]
                id16a = ib[pl.ds(base, 16)]
                id16b = ib[pl.ds(base + 16, 16)]
                rowoff = krow * 1024 + koff
                last = id16b[15]
                has_boundary = last != scr[0]

                def _xrow(j):
                    return plsc.load_gather(
                        xb, [xconst + (rowoff + j)])

                def fast():
                    for t in range(16):
                        off = rowoff + ((t >> 3) * XW + (t & 7) * 128)
                        x1 = xb[pl.ds(off, 16)]
                        x2 = xb[pl.ds(off + 16, 16)]
                        d1 = x1 - e16a
                        d2 = x2 - e16b
                        accT[t, :] = (accT[t, :] + k16a * (d1 * d1)
                                      + k16b * (d2 * d2))

                def slow():
                    collapse()
                    prev_id = scr[0]
                    cnt = scr[1]
                    acc = accr[...]
                    gvec = gvr[...]
                    for j in range(32):
                        x = _xrow(j)
                        kh = k16a if j < 16 else k16b
                        eh = e16a if j < 16 else e16b
                        idh = id16a if j < 16 else id16b
                        d = x - _bcast(eh, j & 15)
                        u = _bcast(kh, j & 15) * (d * d)
                        gid = idh[j & 15]
                        changed = gid != prev_id
                        slot = cnt & 15

                        @pl.when(changed)
                        def _(acc=acc, slot=slot):
                            vb[slot, :] = acc

                        sel = jnp.where(changed, slot, jnp.int32(-1))
                        gvec = jnp.where(lane == sel, prev_id, gvec)
                        flush = changed & (slot == 15)

                        @pl.when(flush)
                        def _(gvec=gvec):
                            pltpu.sync_copy(vb, acc_sh.at[gvec], add=True)

                        gvec = jnp.where(flush, dump_vec, gvec)
                        acc = jnp.where(changed, u, acc + u)
                        cnt = jnp.where(changed, cnt + 1, cnt)
                        prev_id = gid
                    scr[0] = prev_id
                    scr[1] = cnt
                    accr[...] = acc
                    gvr[...] = gvec

                lax.cond(has_boundary, slow, fast)
                return 0

            lax.fori_loop(gstart // 2, NGRP // 2, group, 0)
            collapse()

            slot = scr[1] & 15
            vb[slot, :] = accr[...]
            gvec = jnp.where(lane == slot, scr[0], gvr[...])
            pltpu.sync_copy(vb, acc_sh.at[gvec], add=True)
            gvr[...] = dump_vec

        def run_ntype(x_h, k_h, e_h, i_h):
            start_blk(x_h, k_h, e_h, i_h, 0, 0)

            def pair(i, _):
                j0 = 2 * i
                start_blk(x_h, k_h, e_h, i_h, 1, j0 + 1)
                wait_blk(x_h, k_h, e_h, i_h, 0)
                scan_block(0, j0)
                start_blk(x_h, k_h, e_h, i_h, 0, j0 + 2)
                wait_blk(x_h, k_h, e_h, i_h, 1)
                scan_block(1, j0 + 1)
                return 0

            lax.fori_loop(0, SLOTS // 2, pair, 0)
            wait_blk(x_h, k_h, e_h, i_h, 0)
            scan_block(0, SLOTS - 1)

        run_ntype(x2_h, k2_h, e2_h, i2_h)
        run_ntype(x3_h, k3_h, e3_h, i3_h)

        plsc.subcore_barrier()
        orows = 624
        pltpu.sync_copy(acc_sh.at[pl.ds(s * orows, orows)],
                        out_h.at[c].at[pl.ds(s * orows, orows)])

        @pl.when(s == NS - 1)
        def _():
            pltpu.sync_copy(acc_sh.at[pl.ds(NS * orows, G - NS * orows)],
                            out_h.at[c].at[pl.ds(NS * orows, G - NS * orows)])

    return k(x2, k2, e2, i2, x3, k3, e3, i3)


def _tc_combine(p0, p1):
    def add_k(a_ref, b_ref, o_ref):
        o_ref[...] = a_ref[...] + b_ref[...]

    return pl.pallas_call(
        add_k,
        out_shape=jax.ShapeDtypeStruct((1250, 128), jnp.float32),
    )(p0, p1)


def _xview(x):
    return jnp.transpose(
        x.reshape(NTILE, 128, 2, 8), (2, 0, 3, 1)).reshape(2, NTILE * 1024)


def kernel(x_n2, k_n2, eq_n2, x_n3, k_n3, eq_n3, n2_graph_ids, n3_graph_ids,
           num_graphs):
    del num_graphs
    x2v = _xview(x_n2)
    x3v = _xview(x_n3)
    k2 = k_n2.reshape(NTILE, 128)
    e2 = eq_n2.reshape(NTILE, 128)
    k3 = k_n3.reshape(NTILE, 128)
    e3 = eq_n3.reshape(NTILE, 128)
    i2 = n2_graph_ids.astype(jnp.int32)
    i3 = n3_graph_ids.astype(jnp.int32)
    partials = _sc_partials(x2v, k2, e2, i2, x3v, k3, e3, i3)
    p0 = partials[0].reshape(1250, 128)
    p1 = partials[1].reshape(1250, 128)
    return _tc_combine(p0, p1).reshape(G, T)
